# baseline (speedup 1.0000x reference)
import jax
import jax.numpy as jnp
from jax.experimental import pallas as pl
from jax.experimental.pallas import tpu as pltpu

_N = 16384
_D = 256
_K = 256
_B = 1024


def _lrelu(v):
    return jnp.maximum(v, 0.01 * v)


def _body(x_ref, eps_ref, w0_ref, b0_ref, W1_ref, b1_ref, Wml_ref, bml_ref,
          embT_ref, embA_ref, e2_ref,
          Wd_ref, bd_ref, Wd0_ref, bd0_ref, Wd1_ref, bd1_ref, Wd2_ref, bd2_ref,
          ze_ref, zq_ref, de_ref, dq_ref, cs_ref, ss_ref, tbl_ref):
    def dec(z):
        y = _lrelu(jnp.dot(z, Wd_ref[...]) + bd_ref[...])
        y = _lrelu(jnp.dot(y, Wd0_ref[...]) + bd0_ref[...])
        y = _lrelu(jnp.dot(y, Wd1_ref[...]) + bd1_ref[...])
        y = _lrelu(jnp.dot(y, Wd2_ref[...]) + bd2_ref[...])
        return y

    @pl.when(pl.program_id(0) == 0)
    def _mk_table():
        tbl_ref[...] = dec(embA_ref[:, :_D])
    xb = x_ref[...]
    h = _lrelu(xb * w0_ref[...] + b0_ref[...])
    h = _lrelu(jnp.dot(h, W1_ref[...]) + b1_ref[...])
    ml = jnp.dot(h, Wml_ref[...]) + bml_ref[...]
    mu, lv = ml[:, :_D], ml[:, _D:]
    ze = mu + eps_ref[...] * jnp.exp(0.5 * lv)
    ze_ref[...] = ze

    dots = jnp.dot(ze, embT_ref[...])
    z2 = jnp.sum(ze * ze, axis=1, keepdims=True)
    d = (z2 - 2.0 * dots) + e2_ref[...]
    dmin = jnp.min(d, axis=1, keepdims=True)
    j = jax.lax.broadcasted_iota(jnp.int32, d.shape, 1)
    k = jnp.min(jnp.where(d == dmin, j, _K), axis=1, keepdims=True)

    cs_part = jnp.sum(dmin)

    oh = (j == k).astype(jnp.float32)
    ga = jnp.dot(oh, embA_ref[...])
    zq = ga[:, :_D]
    m = ga[:, _D:]
    ss_part = jnp.sum(m * d)
    zq_ref[...] = zq

    de_ref[...] = dec(ze)[:, 0:1]
    dq_ref[...] = jnp.dot(oh, tbl_ref[...])[:, 0:1]

    @pl.when(pl.program_id(0) == 0)
    def _init():
        cs_ref[...] = jnp.zeros_like(cs_ref)
        ss_ref[...] = jnp.zeros_like(ss_ref)

    cs_ref[...] += cs_part
    ss_ref[...] += ss_part


def kernel(x, W_e0, b_e0, W_e1, b_e1, W_mu, b_mu, W_lv, b_lv,
           W_d, b_d, W_d0, b_d0, W_d1, b_d1, W_d2, b_d2, emb, eps):
    f32 = jnp.float32
    w0p = jnp.zeros((1, 128), f32).at[0, :10].set(W_e0[:, 0])
    b0p = jnp.zeros((1, 128), f32).at[0, :10].set(b_e0)
    W1p = jnp.zeros((128, 128), f32).at[:10, :50].set(W_e1.T)
    b1p = jnp.zeros((1, 128), f32).at[0, :50].set(b_e1)
    Wmlp = (jnp.zeros((128, 2 * _D), f32)
            .at[:50, :_D].set(W_mu.T).at[:50, _D:].set(W_lv.T))
    bmlp = jnp.concatenate([b_mu, b_lv]).reshape(1, 2 * _D)
    embT = emb.T
    e2 = jnp.sum(emb * emb, axis=1).reshape(1, _K)
    idx = jnp.arange(_K, dtype=jnp.int32)
    i1, i2 = idx // 16, idx % 16
    nb = [jnp.clip(i1 - 1, 0, 15) * 16 + i2, jnp.clip(i1 + 1, 0, 15) * 16 + i2,
          i1 * 16 + jnp.clip(i2 - 1, 0, 15), i1 * 16 + jnp.clip(i2 + 1, 0, 15)]
    A = jnp.zeros((_K, _K), f32)
    for nbi in nb:
        A = A.at[idx, nbi].add(1.0)
    embA = jnp.concatenate([emb, A], axis=1)
    Wdp = jnp.zeros((_D, 128), f32).at[:, :100].set(W_d.T)
    bdp = jnp.zeros((1, 128), f32).at[0, :100].set(b_d)
    Wd0p = jnp.zeros((128, 128), f32).at[:100, :60].set(W_d0.T)
    bd0p = jnp.zeros((1, 128), f32).at[0, :60].set(b_d0)
    Wd1p = jnp.zeros((128, 128), f32).at[:60, :30].set(W_d1.T)
    bd1p = jnp.zeros((1, 128), f32).at[0, :30].set(b_d1)
    Wd2p = jnp.zeros((128, 128), f32).at[:30, :1].set(W_d2.T)
    bd2p = jnp.zeros((1, 128), f32).at[0, 0].set(b_d2[0])

    full = lambda shape: pl.BlockSpec(shape, lambda i: (0, 0))
    rows = lambda cols: pl.BlockSpec((_B, cols), lambda i: (i, 0))

    ze, zq, de, dq, cs, ss = pl.pallas_call(
        _body,
        grid=(_N // _B,),
        in_specs=[
            rows(1), rows(_D),
            full((1, 128)), full((1, 128)), full((128, 128)), full((1, 128)),
            full((128, 2 * _D)), full((1, 2 * _D)),
            full((_D, _K)), full((_K, 2 * _K)), full((1, _K)),
            full((_D, 128)), full((1, 128)), full((128, 128)), full((1, 128)),
            full((128, 128)), full((1, 128)), full((128, 128)), full((1, 128)),
        ],
        out_specs=[
            rows(_D), rows(_D), rows(1), rows(1),
            pl.BlockSpec((1, 1), lambda i: (0, 0)),
            pl.BlockSpec((1, 1), lambda i: (0, 0)),
        ],
        out_shape=[
            jax.ShapeDtypeStruct((_N, _D), f32),
            jax.ShapeDtypeStruct((_N, _D), f32),
            jax.ShapeDtypeStruct((_N, 1), f32),
            jax.ShapeDtypeStruct((_N, 1), f32),
            jax.ShapeDtypeStruct((1, 1), f32),
            jax.ShapeDtypeStruct((1, 1), f32),
        ],
        scratch_shapes=[pltpu.VMEM((_K, 128), f32)],
    )(x, eps, w0p, b0p, W1p, b1p, Wmlp, bmlp, embT, embA, e2,
      Wdp, bdp, Wd0p, bd0p, Wd1p, bd1p, Wd2p, bd2p)

    commit_loss = 2.0 * cs[0, 0] / (_N * _D)
    som_loss = ss[0, 0] / (_N * 4 * _D)
    return ze, zq, de, dq, commit_loss, som_loss


# speedup vs baseline: 2.2957x; 2.2957x over previous
import jax
import jax.numpy as jnp
import numpy as np
from jax.experimental import pallas as pl
from jax.experimental.pallas import tpu as pltpu

_N = 16384
_D = 256
_K = 256
_B = 1024


def _make_adj():
    idx = np.arange(_K)
    i1, i2 = idx // 16, idx % 16
    adj = np.zeros((_K, _K), np.float32)
    for nbi in (np.clip(i1 - 1, 0, 15) * 16 + i2, np.clip(i1 + 1, 0, 15) * 16 + i2,
                i1 * 16 + np.clip(i2 - 1, 0, 15), i1 * 16 + np.clip(i2 + 1, 0, 15)):
        np.add.at(adj, (idx, nbi), 1.0)
    return adj


_ADJ = _make_adj()


def _lrelu(v):
    return jnp.maximum(v, 0.01 * v)


def _body(x_ref, eps_ref, w0_ref, b0_ref, W1_ref, b1_ref, Wml_ref, bml_ref,
          embT_ref, embA_ref, e2_ref,
          Wd_ref, bd_ref, Wd0_ref, bd0_ref, Wd1_ref, bd1_ref, Wd2_ref, bd2_ref,
          ze_ref, zq_ref, de_ref, dq_ref, cs_ref, ss_ref, tbl_ref):
    def dec(z):
        y = _lrelu(jnp.dot(z, Wd_ref[...]) + bd_ref[...])
        y = _lrelu(jnp.dot(y, Wd0_ref[...]) + bd0_ref[...])
        y = _lrelu(jnp.dot(y, Wd1_ref[...]) + bd1_ref[...])
        y = _lrelu(jnp.dot(y, Wd2_ref[...]) + bd2_ref[...])
        return y

    @pl.when(pl.program_id(0) == 0)
    def _mk_table():
        tbl_ref[...] = dec(embA_ref[:, :_D])
    xb = x_ref[...]
    h = _lrelu(xb * w0_ref[...] + b0_ref[...])
    h = _lrelu(jnp.dot(h, W1_ref[...]) + b1_ref[...])
    ml = jnp.dot(h, Wml_ref[...]) + bml_ref[...]
    mu, lv = ml[:, :_D], ml[:, _D:]
    ze = mu + eps_ref[...] * jnp.exp(0.5 * lv)
    ze_ref[...] = ze

    dots = jnp.dot(ze, embT_ref[...])
    z2 = jnp.sum(ze * ze, axis=1, keepdims=True)
    d = (z2 - 2.0 * dots) + e2_ref[...]
    dmin = jnp.min(d, axis=1, keepdims=True)
    j = jax.lax.broadcasted_iota(jnp.int32, d.shape, 1)
    k = jnp.min(jnp.where(d == dmin, j, _K), axis=1, keepdims=True)

    cs_part = jnp.sum(dmin)

    oh = (j == k).astype(jnp.float32)
    ga = jnp.dot(oh, embA_ref[...])
    zq = ga[:, :_D]
    m = ga[:, _D:]
    ss_part = jnp.sum(m * d)
    zq_ref[...] = zq

    de_ref[...] = dec(ze)[:, 0:1]
    dq_ref[...] = jnp.dot(oh, tbl_ref[...])[:, 0:1]

    @pl.when(pl.program_id(0) == 0)
    def _init():
        cs_ref[...] = jnp.zeros_like(cs_ref)
        ss_ref[...] = jnp.zeros_like(ss_ref)

    cs_ref[...] += cs_part
    ss_ref[...] += ss_part


def kernel(x, W_e0, b_e0, W_e1, b_e1, W_mu, b_mu, W_lv, b_lv,
           W_d, b_d, W_d0, b_d0, W_d1, b_d1, W_d2, b_d2, emb, eps):
    f32 = jnp.float32
    w0p = jnp.zeros((1, 128), f32).at[0, :10].set(W_e0[:, 0])
    b0p = jnp.zeros((1, 128), f32).at[0, :10].set(b_e0)
    W1p = jnp.zeros((128, 128), f32).at[:10, :50].set(W_e1.T)
    b1p = jnp.zeros((1, 128), f32).at[0, :50].set(b_e1)
    Wmlp = (jnp.zeros((128, 2 * _D), f32)
            .at[:50, :_D].set(W_mu.T).at[:50, _D:].set(W_lv.T))
    bmlp = jnp.concatenate([b_mu, b_lv]).reshape(1, 2 * _D)
    embT = emb.T
    e2 = jnp.sum(emb * emb, axis=1).reshape(1, _K)
    embA = jnp.concatenate([emb, jnp.asarray(_ADJ)], axis=1)
    Wdp = jnp.zeros((_D, 128), f32).at[:, :100].set(W_d.T)
    bdp = jnp.zeros((1, 128), f32).at[0, :100].set(b_d)
    Wd0p = jnp.zeros((128, 128), f32).at[:100, :60].set(W_d0.T)
    bd0p = jnp.zeros((1, 128), f32).at[0, :60].set(b_d0)
    Wd1p = jnp.zeros((128, 128), f32).at[:60, :30].set(W_d1.T)
    bd1p = jnp.zeros((1, 128), f32).at[0, :30].set(b_d1)
    Wd2p = jnp.zeros((128, 128), f32).at[:30, :1].set(W_d2.T)
    bd2p = jnp.zeros((1, 128), f32).at[0, 0].set(b_d2[0])

    full = lambda shape: pl.BlockSpec(shape, lambda i: (0, 0))
    rows = lambda cols: pl.BlockSpec((_B, cols), lambda i: (i, 0))

    ze, zq, de, dq, cs, ss = pl.pallas_call(
        _body,
        grid=(_N // _B,),
        in_specs=[
            rows(1), rows(_D),
            full((1, 128)), full((1, 128)), full((128, 128)), full((1, 128)),
            full((128, 2 * _D)), full((1, 2 * _D)),
            full((_D, _K)), full((_K, 2 * _K)), full((1, _K)),
            full((_D, 128)), full((1, 128)), full((128, 128)), full((1, 128)),
            full((128, 128)), full((1, 128)), full((128, 128)), full((1, 128)),
        ],
        out_specs=[
            rows(_D), rows(_D), rows(1), rows(1),
            pl.BlockSpec((1, 1), lambda i: (0, 0)),
            pl.BlockSpec((1, 1), lambda i: (0, 0)),
        ],
        out_shape=[
            jax.ShapeDtypeStruct((_N, _D), f32),
            jax.ShapeDtypeStruct((_N, _D), f32),
            jax.ShapeDtypeStruct((_N, 1), f32),
            jax.ShapeDtypeStruct((_N, 1), f32),
            jax.ShapeDtypeStruct((1, 1), f32),
            jax.ShapeDtypeStruct((1, 1), f32),
        ],
        scratch_shapes=[pltpu.VMEM((_K, 128), f32)],
    )(x, eps, w0p, b0p, W1p, b1p, Wmlp, bmlp, embT, embA, e2,
      Wdp, bdp, Wd0p, bd0p, Wd1p, bd1p, Wd2p, bd2p)

    commit_loss = 2.0 * cs[0, 0] / (_N * _D)
    som_loss = ss[0, 0] / (_N * 4 * _D)
    return ze, zq, de, dq, commit_loss, som_loss
